# baseline (speedup 1.0000x reference)
import functools

import jax
import jax.numpy as jnp
from jax import lax
from jax.experimental import pallas as pl
from jax.experimental.pallas import tpu as pltpu
from jax.experimental.pallas import tpu_sc as plsc

N = 10000
E = 320000
AA = 21
D1 = 149
D2 = 298
D3 = 596
OUT_DIM = 486
NG = 64

NP = 10240
EP = 327680
CW = 64
CK = 128
NSUB = 16
RT = EP // CK
RH = RT // 2
ZR = NP // NSUB
BR = 512
GRID = NP // BR
T1 = 3
T2 = 5
T3 = 10

_HIGH = lax.Precision.HIGHEST


@functools.cache
def _mesh():
    return plsc.VectorSubcoreMesh(core_axis_name="c", subcore_axis_name="s",
                                  num_cores=2, num_subcores=NSUB)


@functools.cache
def _make_sc_agg(rows_per_core):
    rw = rows_per_core // NSUB

    @functools.partial(
        pl.kernel,
        out_type=jax.ShapeDtypeStruct((2 * NP, CW), jnp.float32),
        mesh=_mesh(),
        compiler_params=pltpu.CompilerParams(use_tc_tiling_on_sc=False),
        scratch_types=[
            pltpu.VMEM((rw, CK), jnp.int32),
            pltpu.VMEM((rw, CK), jnp.int32),
            pltpu.VMEM((5, CK, CW), jnp.float32),
            pltpu.VMEM((CK, CW), jnp.float32),
            pltpu.VMEM_SHARED((NP, CW), jnp.float32),
            [pltpu.SemaphoreType.DMA] * 5,
            [pltpu.SemaphoreType.DMA] * 5,
        ],
    )
    def agg(table, src_idx, dst_idx, out, src_v, dst_v, ring, zbuf,
            acc, gsems, ssems):
        c = lax.axis_index("c")
        s = lax.axis_index("s")
        bufs = [ring.at[b] for b in range(5)]

        def zrow(i, carry):
            for o in range(0, CW, 16):
                zbuf[i, pl.ds(o, 16)] = jnp.zeros((16,), jnp.float32)
            return carry
        lax.fori_loop(0, CK, zrow, 0)
        for r in range(ZR // CK):
            pltpu.sync_copy(zbuf, acc.at[pl.ds(s * ZR + r * CK, CK)])

        base = c * rows_per_core + s * rw
        pltpu.sync_copy(src_idx.at[pl.ds(base, rw)], src_v)
        pltpu.sync_copy(dst_idx.at[pl.ds(base, rw)], dst_v)
        plsc.subcore_barrier()

        for b in range(2):
            pltpu.async_copy(table.at[src_v.at[b]], bufs[b], gsems[b])

        def penta(jj, carry):
            for b in range(5):
                j = jj * 5 + b
                pltpu.make_async_copy(table.at[src_v.at[j]], bufs[b],
                                      gsems[b]).wait()
                pltpu.async_copy(bufs[b], acc.at[dst_v.at[j]], ssems[b],
                                 add=True)
                bp = (b + 2) % 5

                @pl.when(j >= 3)
                def _():
                    pltpu.make_async_copy(bufs[bp], acc.at[dst_v.at[j - 3]],
                                          ssems[bp]).wait()

                @pl.when(j + 2 < rw)
                def _():
                    pltpu.async_copy(table.at[src_v.at[j + 2]], bufs[bp],
                                     gsems[bp])
            return carry
        lax.fori_loop(0, rw // 5, penta, 0)
        for j in (rw - 3, rw - 2, rw - 1):
            pltpu.make_async_copy(bufs[j % 5], acc.at[dst_v.at[j]],
                                  ssems[j % 5]).wait()

        plsc.subcore_barrier()
        pltpu.sync_copy(acc.at[pl.ds(s * ZR, ZR)],
                        out.at[pl.ds(c * NP + s * ZR, ZR)])

    return agg


def _sc_agg_cs(table, src_idx, dst_idx):
    return _make_sc_agg(RT)(table, src_idx, dst_idx).reshape(2, NP, CW)


def _sc_agg_es(table, src_idx, dst_idx):
    return _make_sc_agg(RH)(table, src_idx, dst_idx).reshape(2, NP, CW)


@functools.cache
def _make_sc_deg():
    rw = RH // NSUB

    @functools.partial(
        pl.kernel,
        out_type=jax.ShapeDtypeStruct((2 * NP, 16), jnp.float32),
        mesh=_mesh(),
        compiler_params=pltpu.CompilerParams(use_tc_tiling_on_sc=False),
        scratch_types=[
            pltpu.VMEM((RH // NSUB, CK), jnp.int32),
            pltpu.VMEM((CK, 16), jnp.float32),
            pltpu.VMEM((CK, 16), jnp.float32),
            pltpu.VMEM_SHARED((NP, 16), jnp.float32),
        ],
    )
    def deg(dst_idx, out, dst_v, ones_v, zbuf, acc):
        c = lax.axis_index("c")
        s = lax.axis_index("s")

        def frow(i, carry):
            zbuf[i, :] = jnp.zeros((16,), jnp.float32)
            ones_v[i, :] = jnp.ones((16,), jnp.float32)
            return carry
        lax.fori_loop(0, CK, frow, 0)
        for r in range(ZR // CK):
            pltpu.sync_copy(zbuf, acc.at[pl.ds(s * ZR + r * CK, CK)])

        pltpu.sync_copy(dst_idx.at[pl.ds(c * RH + s * rw, rw)], dst_v)
        plsc.subcore_barrier()

        def chunk(j, carry):
            pltpu.sync_copy(ones_v, acc.at[dst_v.at[j]], add=True)
            return carry
        lax.fori_loop(0, rw, chunk, 0)

        plsc.subcore_barrier()
        pltpu.sync_copy(acc.at[pl.ds(s * ZR, ZR)],
                        out.at[pl.ds(c * NP + s * ZR, ZR)])

    return deg


def _sc_deg(dst_idx):
    return _make_sc_deg()(dst_idx).reshape(2, NP, 16)


def _rsqrt(x):
    r = lax.rsqrt(x)
    return r * (1.5 - 0.5 * x * r * r)


def _dinv_of(deg_ref):
    deg = deg_ref[0, :, 0:1] + deg_ref[1, :, 0:1] + 1.0
    return _rsqrt(deg)


def _mmd(a, b):
    return jnp.dot(a, b, preferred_element_type=jnp.float32)


def _write_tiles(out_ref, g):
    for t in range(out_ref.shape[0]):
        out_ref[t, :, :] = g[:, t * CW:(t + 1) * CW]


def _cat_tiles(tiles):
    return jnp.concatenate(tiles, axis=1)


def _k0_body(x1_ref, x2_ref, deg_ref, w1_ref, w2_ref, b_ref, wc_ref,
             out_ref):
    dinv = _dinv_of(deg_ref)
    m = _mmd(x2_ref[...], w2_ref[...]) + _mmd(x1_ref[...], w1_ref[...])
    h0 = jnp.maximum(m + b_ref[0:1, :], 0.0)
    hw = _mmd(h0, wc_ref[...])
    _write_tiles(out_ref, dinv * hw)


def _conv_body(n_in, s_refs_cs, s_es_ref, g_ref, deg_ref, b_ref, wc_ref,
               out_ref):
    dinv = _dinv_of(deg_ref)
    s_tiles = []
    for r in s_refs_cs:
        s_tiles.append(r[0])
        s_tiles.append(r[1])
    if s_es_ref is not None:
        s_tiles.append(s_es_ref[0] + s_es_ref[1])
    u = _cat_tiles([dinv * (s_tiles[t] + g_ref[t]) for t in range(n_in)])
    h = jnp.maximum(u + b_ref[0:1, :], 0.0)
    hw = _mmd(h, wc_ref[...])
    _write_tiles(out_ref, dinv * hw)


def _k1_body(sa, se, g_ref, deg_ref, b_ref, wc_ref, out_ref):
    _conv_body(T1, [sa], se, g_ref, deg_ref, b_ref, wc_ref, out_ref)


def _k2_body(sa, sb, se, g_ref, deg_ref, b_ref, wc_ref, out_ref):
    _conv_body(T2, [sa, sb], se, g_ref, deg_ref, b_ref, wc_ref, out_ref)


def _k3_body(sa, sb, sc, sd, sf, g_ref, deg_ref, b_ref, p_ref, out_ref):
    dinv = _dinv_of(deg_ref)
    s_tiles = []
    for r in (sa, sb, sc, sd, sf):
        s_tiles.append(r[0])
        s_tiles.append(r[1])
    u = _cat_tiles([dinv * (s_tiles[t] + g_ref[t]) for t in range(T3)])
    h3 = jnp.maximum(u + b_ref[0:1, :], 0.0)
    part = lax.dot_general(p_ref[...], h3, (((0,), (0,)), ((), ())),
                           precision=_HIGH,
                           preferred_element_type=jnp.float32)

    @pl.when(pl.program_id(0) == 0)
    def _():
        out_ref[...] = jnp.zeros_like(out_ref)

    out_ref[...] += part


def _k4_body(pool_ref, w1_ref, b1_ref, gm_ref, bt_ref, w2_ref, b2_ref,
             out_ref):
    pool = pool_ref[...]
    cnt = pool[:, D3:D3 + 1]
    pooled = pool / jnp.maximum(cnt, 1.0)
    z = _mmd(pooled, w1_ref[...]) + b1_ref[0:1, :]
    mu = jnp.mean(z, axis=0, keepdims=True)
    zc = z - mu
    var = jnp.mean(zc * zc, axis=0, keepdims=True)
    zn = zc / jnp.sqrt(var + 1e-5) * gm_ref[0:1, :] + bt_ref[0:1, :]
    r = jnp.maximum(zn, 0.0)
    z2 = _mmd(r, w2_ref[...]) + b2_ref[0:1, :]
    out_ref[...] = 1.0 / (1.0 + jnp.exp(-z2))


def _row_spec(cols):
    return pl.BlockSpec((BR, cols), lambda i: (i, 0))


def _tile_spec(t):
    return pl.BlockSpec((t, BR, CW), lambda i: (0, i, 0))


def _deg_spec():
    return pl.BlockSpec((2, BR, 16), lambda i: (0, i, 0))


def _full_spec(shape):
    nd = len(shape)
    return pl.BlockSpec(shape, lambda i: (0,) * nd)


def _k0_call(x1, x2, deg2, w1p, w2p, b0, wc1):
    return pl.pallas_call(
        _k0_body,
        grid=(GRID,),
        in_specs=[_row_spec(112), _row_spec(24), _deg_spec(),
                  _full_spec((112, T1 * CW)), _full_spec((24, T1 * CW)),
                  _full_spec((8, T1 * CW)), _full_spec((T1 * CW, T1 * CW))],
        out_specs=_tile_spec(T1),
        out_shape=jax.ShapeDtypeStruct((T1, NP, CW), jnp.float32),
    )(x1, x2, deg2, w1p, w2p, b0, wc1)


def _k1_call(sa, se, g, deg2, b, wc):
    return pl.pallas_call(
        _k1_body,
        grid=(GRID,),
        in_specs=[_tile_spec(2), _tile_spec(2), _tile_spec(T1), _deg_spec(),
                  _full_spec((8, T1 * CW)), _full_spec(wc.shape)],
        out_specs=_tile_spec(T2),
        out_shape=jax.ShapeDtypeStruct((T2, NP, CW), jnp.float32),
    )(sa, se, g, deg2, b, wc)


def _k2_call(sa, sb, se, g, deg2, b, wc):
    return pl.pallas_call(
        _k2_body,
        grid=(GRID,),
        in_specs=[_tile_spec(2), _tile_spec(2), _tile_spec(2),
                  _tile_spec(T2), _deg_spec(),
                  _full_spec((8, T2 * CW)), _full_spec(wc.shape)],
        out_specs=_tile_spec(T3),
        out_shape=jax.ShapeDtypeStruct((T3, NP, CW), jnp.float32),
    )(sa, sb, se, g, deg2, b, wc)


def _k3_call(s_list, g, deg2, b, p):
    return pl.pallas_call(
        _k3_body,
        grid=(GRID,),
        in_specs=[_tile_spec(2)] * 5 + [_tile_spec(T3), _deg_spec(),
                                        _full_spec((8, T3 * CW)),
                                        _row_spec(NG)],
        out_specs=_full_spec((NG, T3 * CW)),
        out_shape=jax.ShapeDtypeStruct((NG, T3 * CW), jnp.float32),
    )(*s_list, g, deg2, b, p)


def _k4_call(pool, w1, b1, gm, bt, w2, b2):
    return pl.pallas_call(
        _k4_body,
        grid=(1,),
        in_specs=[_full_spec((NG, T3 * CW)), _full_spec((T3 * CW, 1024)),
                  _full_spec((8, 1024)), _full_spec((8, 1024)),
                  _full_spec((8, 1024)), _full_spec((1024, 512)),
                  _full_spec((8, 512))],
        out_specs=_full_spec((NG, 512)),
        out_shape=jax.ShapeDtypeStruct((NG, 512), jnp.float32),
    )(pool, w1, b1, gm, bt, w2, b2)


def _tile8(v, cols):
    row = jnp.zeros((cols,), jnp.float32).at[:v.shape[0]].set(v)
    return jnp.tile(row[None, :], (8, 1))


def kernel(x, edge_index, batch, W1, b1, W2, b2, Wc1, bc1, Wc2, bc2, Wc3, bc3,
           Wf1, bf1, gamma, beta, Wf2, bf2):
    f32 = jnp.float32

    x1 = jnp.zeros((NP, 112), f32).at[:N, :107].set(x[:, AA:])
    x2 = jnp.zeros((NP, 24), f32).at[:N, :AA].set(x[:, :AA])
    batch_p = jnp.concatenate(
        [batch.astype(jnp.int32), jnp.full((NP - N,), NG, jnp.int32)])
    onehot = (batch_p[:, None] == jnp.arange(NG, dtype=jnp.int32)[None, :]
              ).astype(f32)

    pad = jnp.full((EP - E,), N, jnp.int32)
    srcp = jnp.concatenate([edge_index[0].astype(jnp.int32), pad])
    dstp = jnp.concatenate([edge_index[1].astype(jnp.int32), pad])
    src2d = srcp.reshape(RT, CK)
    dst2d = dstp.reshape(RT, CK)

    def cs_idx(t):
        return jnp.concatenate([src2d + t * NP, src2d + (t + 1) * NP])

    dst2 = jnp.concatenate([dst2d, dst2d])

    w1p = jnp.zeros((112, T1 * CW), f32).at[:107, AA:D1].set(W1)
    w2p = jnp.zeros((24, T1 * CW), f32).at[:AA, :AA].set(W2)
    b0 = _tile8(jnp.concatenate([b2, b1]), T1 * CW)
    wc1 = jnp.zeros((T1 * CW, T1 * CW), f32).at[:D1, :D1].set(Wc1)
    bc1p = _tile8(bc1, T1 * CW)
    wc2 = jnp.zeros((T1 * CW, T2 * CW), f32).at[:D1, :D2].set(Wc2)
    bc2p = _tile8(bc2, T2 * CW)
    wc3 = jnp.zeros((T2 * CW, T3 * CW), f32).at[:D2, :D3].set(Wc3)
    bc3p = _tile8(bc3, T3 * CW).at[:, D3].set(1.0)
    wf1 = jnp.zeros((T3 * CW, 1024), f32).at[:D3, :].set(Wf1)
    bf1p = _tile8(bf1, 1024)
    gmp = _tile8(gamma, 1024)
    btp = _tile8(beta, 1024)
    wf2 = jnp.zeros((1024, 512), f32).at[:, :OUT_DIM].set(Wf2)
    bf2p = _tile8(bf2, 512)

    deg2 = _sc_deg(dst2d)
    g1 = _k0_call(x1, x2, deg2, w1p, w2p, b0, wc1)
    t1 = g1.reshape(T1 * NP, CW)
    s1a = _sc_agg_cs(t1, cs_idx(0), dst2)
    s1e = _sc_agg_es(t1, src2d + 2 * NP, dst2d)
    g2 = _k1_call(s1a, s1e, g1, deg2, bc1p, wc2)
    t2 = g2.reshape(T2 * NP, CW)
    s2a = _sc_agg_cs(t2, cs_idx(0), dst2)
    s2b = _sc_agg_cs(t2, cs_idx(2), dst2)
    s2e = _sc_agg_es(t2, src2d + 4 * NP, dst2d)
    g3 = _k2_call(s2a, s2b, s2e, g2, deg2, bc2p, wc3)
    t3 = g3.reshape(T3 * NP, CW)
    s3 = [_sc_agg_cs(t3, cs_idx(2 * i), dst2) for i in range(5)]
    pool = _k3_call(s3, g3, deg2, bc3p, onehot)
    out = _k4_call(pool, wf1, bf1p, gmp, btp, wf2, bf2p)
    return out[:, :OUT_DIM]


# speedup vs baseline: 1.0268x; 1.0268x over previous
import functools

import jax
import jax.numpy as jnp
from jax import lax
from jax.experimental import pallas as pl
from jax.experimental.pallas import tpu as pltpu
from jax.experimental.pallas import tpu_sc as plsc

N = 10000
E = 320000
AA = 21
D1 = 149
D2 = 298
D3 = 596
OUT_DIM = 486
NG = 64

NP = 10240
EP = 327680
CW = 64
CK = 128
NSUB = 16
RT = EP // CK
RH = RT // 2
ZR = NP // NSUB
BR = 512
GRID = NP // BR
T1 = 3
T2 = 5
T3 = 10

_HIGH = lax.Precision.HIGHEST


@functools.cache
def _mesh():
    return plsc.VectorSubcoreMesh(core_axis_name="c", subcore_axis_name="s",
                                  num_cores=2, num_subcores=NSUB)


@functools.cache
def _make_sc_agg(rows_per_core):
    rw = rows_per_core // NSUB

    @functools.partial(
        pl.kernel,
        out_type=jax.ShapeDtypeStruct((2 * NP, CW), jnp.float32),
        mesh=_mesh(),
        compiler_params=pltpu.CompilerParams(use_tc_tiling_on_sc=False),
        scratch_types=[
            pltpu.VMEM((rw, CK), jnp.int32),
            pltpu.VMEM((rw, CK), jnp.int32),
            pltpu.VMEM((5, CK, CW), jnp.float32),
            pltpu.VMEM((CK, CW), jnp.float32),
            pltpu.VMEM_SHARED((NP, CW), jnp.float32),
            [pltpu.SemaphoreType.DMA] * 5,
            [pltpu.SemaphoreType.DMA] * 5,
        ],
    )
    def agg(table, src_idx, dst_idx, out, src_v, dst_v, ring, zbuf,
            acc, gsems, ssems):
        c = lax.axis_index("c")
        s = lax.axis_index("s")
        bufs = [ring.at[b] for b in range(5)]

        def zrow(i, carry):
            for o in range(0, CW, 16):
                zbuf[i, pl.ds(o, 16)] = jnp.zeros((16,), jnp.float32)
            return carry
        lax.fori_loop(0, CK, zrow, 0)
        for r in range(ZR // CK):
            pltpu.sync_copy(zbuf, acc.at[pl.ds(s * ZR + r * CK, CK)])

        base = c * rows_per_core + s * rw
        pltpu.sync_copy(src_idx.at[pl.ds(base, rw)], src_v)
        pltpu.sync_copy(dst_idx.at[pl.ds(base, rw)], dst_v)
        plsc.subcore_barrier()

        for b in range(3):
            pltpu.async_copy(table.at[src_v.at[b]], bufs[b], gsems[b])

        def penta(jj, carry):
            for b in range(5):
                j = jj * 5 + b
                pltpu.make_async_copy(table.at[src_v.at[j]], bufs[b],
                                      gsems[b]).wait()
                pltpu.async_copy(bufs[b], acc.at[dst_v.at[j]], ssems[b],
                                 add=True)
                bp = (b + 3) % 5

                @pl.when(j >= 2)
                def _():
                    pltpu.make_async_copy(bufs[bp], acc.at[dst_v.at[j - 2]],
                                          ssems[bp]).wait()

                @pl.when(j + 3 < rw)
                def _():
                    pltpu.async_copy(table.at[src_v.at[j + 3]], bufs[bp],
                                     gsems[bp])
            return carry
        lax.fori_loop(0, rw // 5, penta, 0)
        for j in (rw - 2, rw - 1):
            pltpu.make_async_copy(bufs[j % 5], acc.at[dst_v.at[j]],
                                  ssems[j % 5]).wait()

        plsc.subcore_barrier()
        pltpu.sync_copy(acc.at[pl.ds(s * ZR, ZR)],
                        out.at[pl.ds(c * NP + s * ZR, ZR)])

    return agg


def _sc_agg_cs(table, src_idx, dst_idx):
    return _make_sc_agg(RT)(table, src_idx, dst_idx).reshape(2, NP, CW)


def _sc_agg_es(table, src_idx, dst_idx):
    return _make_sc_agg(RH)(table, src_idx, dst_idx).reshape(2, NP, CW)


@functools.cache
def _make_sc_deg():
    rw = RH // NSUB

    @functools.partial(
        pl.kernel,
        out_type=jax.ShapeDtypeStruct((2 * NP, 16), jnp.float32),
        mesh=_mesh(),
        compiler_params=pltpu.CompilerParams(use_tc_tiling_on_sc=False),
        scratch_types=[
            pltpu.VMEM((RH // NSUB, CK), jnp.int32),
            pltpu.VMEM((CK, 16), jnp.float32),
            pltpu.VMEM((CK, 16), jnp.float32),
            pltpu.VMEM_SHARED((NP, 16), jnp.float32),
        ],
    )
    def deg(dst_idx, out, dst_v, ones_v, zbuf, acc):
        c = lax.axis_index("c")
        s = lax.axis_index("s")

        def frow(i, carry):
            zbuf[i, :] = jnp.zeros((16,), jnp.float32)
            ones_v[i, :] = jnp.ones((16,), jnp.float32)
            return carry
        lax.fori_loop(0, CK, frow, 0)
        for r in range(ZR // CK):
            pltpu.sync_copy(zbuf, acc.at[pl.ds(s * ZR + r * CK, CK)])

        pltpu.sync_copy(dst_idx.at[pl.ds(c * RH + s * rw, rw)], dst_v)
        plsc.subcore_barrier()

        def chunk(j, carry):
            pltpu.sync_copy(ones_v, acc.at[dst_v.at[j]], add=True)
            return carry
        lax.fori_loop(0, rw, chunk, 0)

        plsc.subcore_barrier()
        pltpu.sync_copy(acc.at[pl.ds(s * ZR, ZR)],
                        out.at[pl.ds(c * NP + s * ZR, ZR)])

    return deg


def _sc_deg(dst_idx):
    return _make_sc_deg()(dst_idx).reshape(2, NP, 16)


def _rsqrt(x):
    r = lax.rsqrt(x)
    return r * (1.5 - 0.5 * x * r * r)


def _dinv_of(deg_ref):
    deg = deg_ref[0, :, 0:1] + deg_ref[1, :, 0:1] + 1.0
    return _rsqrt(deg)


def _mmd(a, b):
    return jnp.dot(a, b, preferred_element_type=jnp.float32)


def _write_tiles(out_ref, g):
    for t in range(out_ref.shape[0]):
        out_ref[t, :, :] = g[:, t * CW:(t + 1) * CW]


def _cat_tiles(tiles):
    return jnp.concatenate(tiles, axis=1)


def _k0_body(x1_ref, x2_ref, deg_ref, w1_ref, w2_ref, b_ref, wc_ref,
             out_ref):
    dinv = _dinv_of(deg_ref)
    m = _mmd(x2_ref[...], w2_ref[...]) + _mmd(x1_ref[...], w1_ref[...])
    h0 = jnp.maximum(m + b_ref[0:1, :], 0.0)
    hw = _mmd(h0, wc_ref[...])
    _write_tiles(out_ref, dinv * hw)


def _conv_body(n_in, s_refs_cs, s_es_ref, g_ref, deg_ref, b_ref, wc_ref,
               out_ref):
    dinv = _dinv_of(deg_ref)
    s_tiles = []
    for r in s_refs_cs:
        s_tiles.append(r[0])
        s_tiles.append(r[1])
    if s_es_ref is not None:
        s_tiles.append(s_es_ref[0] + s_es_ref[1])
    u = _cat_tiles([dinv * (s_tiles[t] + g_ref[t]) for t in range(n_in)])
    h = jnp.maximum(u + b_ref[0:1, :], 0.0)
    hw = _mmd(h, wc_ref[...])
    _write_tiles(out_ref, dinv * hw)


def _k1_body(sa, se, g_ref, deg_ref, b_ref, wc_ref, out_ref):
    _conv_body(T1, [sa], se, g_ref, deg_ref, b_ref, wc_ref, out_ref)


def _k2_body(sa, sb, se, g_ref, deg_ref, b_ref, wc_ref, out_ref):
    _conv_body(T2, [sa, sb], se, g_ref, deg_ref, b_ref, wc_ref, out_ref)


def _k3_body(sa, sb, sc, sd, sf, g_ref, deg_ref, b_ref, p_ref, out_ref):
    dinv = _dinv_of(deg_ref)
    s_tiles = []
    for r in (sa, sb, sc, sd, sf):
        s_tiles.append(r[0])
        s_tiles.append(r[1])
    u = _cat_tiles([dinv * (s_tiles[t] + g_ref[t]) for t in range(T3)])
    h3 = jnp.maximum(u + b_ref[0:1, :], 0.0)
    part = lax.dot_general(p_ref[...], h3, (((0,), (0,)), ((), ())),
                           precision=_HIGH,
                           preferred_element_type=jnp.float32)

    @pl.when(pl.program_id(0) == 0)
    def _():
        out_ref[...] = jnp.zeros_like(out_ref)

    out_ref[...] += part


def _k4_body(pool_ref, w1_ref, b1_ref, gm_ref, bt_ref, w2_ref, b2_ref,
             out_ref):
    pool = pool_ref[...]
    cnt = pool[:, D3:D3 + 1]
    pooled = pool / jnp.maximum(cnt, 1.0)
    z = _mmd(pooled, w1_ref[...]) + b1_ref[0:1, :]
    mu = jnp.mean(z, axis=0, keepdims=True)
    zc = z - mu
    var = jnp.mean(zc * zc, axis=0, keepdims=True)
    zn = zc / jnp.sqrt(var + 1e-5) * gm_ref[0:1, :] + bt_ref[0:1, :]
    r = jnp.maximum(zn, 0.0)
    z2 = _mmd(r, w2_ref[...]) + b2_ref[0:1, :]
    out_ref[...] = 1.0 / (1.0 + jnp.exp(-z2))


def _row_spec(cols):
    return pl.BlockSpec((BR, cols), lambda i: (i, 0))


def _tile_spec(t):
    return pl.BlockSpec((t, BR, CW), lambda i: (0, i, 0))


def _deg_spec():
    return pl.BlockSpec((2, BR, 16), lambda i: (0, i, 0))


def _full_spec(shape):
    nd = len(shape)
    return pl.BlockSpec(shape, lambda i: (0,) * nd)


def _k0_call(x1, x2, deg2, w1p, w2p, b0, wc1):
    return pl.pallas_call(
        _k0_body,
        grid=(GRID,),
        in_specs=[_row_spec(112), _row_spec(24), _deg_spec(),
                  _full_spec((112, T1 * CW)), _full_spec((24, T1 * CW)),
                  _full_spec((8, T1 * CW)), _full_spec((T1 * CW, T1 * CW))],
        out_specs=_tile_spec(T1),
        out_shape=jax.ShapeDtypeStruct((T1, NP, CW), jnp.float32),
    )(x1, x2, deg2, w1p, w2p, b0, wc1)


def _k1_call(sa, se, g, deg2, b, wc):
    return pl.pallas_call(
        _k1_body,
        grid=(GRID,),
        in_specs=[_tile_spec(2), _tile_spec(2), _tile_spec(T1), _deg_spec(),
                  _full_spec((8, T1 * CW)), _full_spec(wc.shape)],
        out_specs=_tile_spec(T2),
        out_shape=jax.ShapeDtypeStruct((T2, NP, CW), jnp.float32),
    )(sa, se, g, deg2, b, wc)


def _k2_call(sa, sb, se, g, deg2, b, wc):
    return pl.pallas_call(
        _k2_body,
        grid=(GRID,),
        in_specs=[_tile_spec(2), _tile_spec(2), _tile_spec(2),
                  _tile_spec(T2), _deg_spec(),
                  _full_spec((8, T2 * CW)), _full_spec(wc.shape)],
        out_specs=_tile_spec(T3),
        out_shape=jax.ShapeDtypeStruct((T3, NP, CW), jnp.float32),
    )(sa, sb, se, g, deg2, b, wc)


def _k3_call(s_list, g, deg2, b, p):
    return pl.pallas_call(
        _k3_body,
        grid=(GRID,),
        in_specs=[_tile_spec(2)] * 5 + [_tile_spec(T3), _deg_spec(),
                                        _full_spec((8, T3 * CW)),
                                        _row_spec(NG)],
        out_specs=_full_spec((NG, T3 * CW)),
        out_shape=jax.ShapeDtypeStruct((NG, T3 * CW), jnp.float32),
    )(*s_list, g, deg2, b, p)


def _k4_call(pool, w1, b1, gm, bt, w2, b2):
    return pl.pallas_call(
        _k4_body,
        grid=(1,),
        in_specs=[_full_spec((NG, T3 * CW)), _full_spec((T3 * CW, 1024)),
                  _full_spec((8, 1024)), _full_spec((8, 1024)),
                  _full_spec((8, 1024)), _full_spec((1024, 512)),
                  _full_spec((8, 512))],
        out_specs=_full_spec((NG, 512)),
        out_shape=jax.ShapeDtypeStruct((NG, 512), jnp.float32),
    )(pool, w1, b1, gm, bt, w2, b2)


def _tile8(v, cols):
    row = jnp.zeros((cols,), jnp.float32).at[:v.shape[0]].set(v)
    return jnp.tile(row[None, :], (8, 1))


def kernel(x, edge_index, batch, W1, b1, W2, b2, Wc1, bc1, Wc2, bc2, Wc3, bc3,
           Wf1, bf1, gamma, beta, Wf2, bf2):
    f32 = jnp.float32

    x1 = jnp.zeros((NP, 112), f32).at[:N, :107].set(x[:, AA:])
    x2 = jnp.zeros((NP, 24), f32).at[:N, :AA].set(x[:, :AA])
    batch_p = jnp.concatenate(
        [batch.astype(jnp.int32), jnp.full((NP - N,), NG, jnp.int32)])
    onehot = (batch_p[:, None] == jnp.arange(NG, dtype=jnp.int32)[None, :]
              ).astype(f32)

    pad = jnp.full((EP - E,), N, jnp.int32)
    srcp = jnp.concatenate([edge_index[0].astype(jnp.int32), pad])
    dstp = jnp.concatenate([edge_index[1].astype(jnp.int32), pad])
    src2d = srcp.reshape(RT, CK)
    dst2d = dstp.reshape(RT, CK)

    def cs_idx(t):
        return jnp.concatenate([src2d + t * NP, src2d + (t + 1) * NP])

    dst2 = jnp.concatenate([dst2d, dst2d])

    w1p = jnp.zeros((112, T1 * CW), f32).at[:107, AA:D1].set(W1)
    w2p = jnp.zeros((24, T1 * CW), f32).at[:AA, :AA].set(W2)
    b0 = _tile8(jnp.concatenate([b2, b1]), T1 * CW)
    wc1 = jnp.zeros((T1 * CW, T1 * CW), f32).at[:D1, :D1].set(Wc1)
    bc1p = _tile8(bc1, T1 * CW)
    wc2 = jnp.zeros((T1 * CW, T2 * CW), f32).at[:D1, :D2].set(Wc2)
    bc2p = _tile8(bc2, T2 * CW)
    wc3 = jnp.zeros((T2 * CW, T3 * CW), f32).at[:D2, :D3].set(Wc3)
    bc3p = _tile8(bc3, T3 * CW).at[:, D3].set(1.0)
    wf1 = jnp.zeros((T3 * CW, 1024), f32).at[:D3, :].set(Wf1)
    bf1p = _tile8(bf1, 1024)
    gmp = _tile8(gamma, 1024)
    btp = _tile8(beta, 1024)
    wf2 = jnp.zeros((1024, 512), f32).at[:, :OUT_DIM].set(Wf2)
    bf2p = _tile8(bf2, 512)

    deg2 = _sc_deg(dst2d)
    g1 = _k0_call(x1, x2, deg2, w1p, w2p, b0, wc1)
    t1 = g1.reshape(T1 * NP, CW)
    s1a = _sc_agg_cs(t1, cs_idx(0), dst2)
    s1e = _sc_agg_es(t1, src2d + 2 * NP, dst2d)
    g2 = _k1_call(s1a, s1e, g1, deg2, bc1p, wc2)
    t2 = g2.reshape(T2 * NP, CW)
    s2a = _sc_agg_cs(t2, cs_idx(0), dst2)
    s2b = _sc_agg_cs(t2, cs_idx(2), dst2)
    s2e = _sc_agg_es(t2, src2d + 4 * NP, dst2d)
    g3 = _k2_call(s2a, s2b, s2e, g2, deg2, bc2p, wc3)
    t3 = g3.reshape(T3 * NP, CW)
    s3 = [_sc_agg_cs(t3, cs_idx(2 * i), dst2) for i in range(5)]
    pool = _k3_call(s3, g3, deg2, bc3p, onehot)
    out = _k4_call(pool, wf1, bf1p, gmp, btp, wf2, bf2p)
    return out[:, :OUT_DIM]
